# baseline (speedup 1.0000x reference)
import functools

import jax
import jax.numpy as jnp
from jax import lax
from jax.experimental import pallas as pl
from jax.experimental.pallas import tpu as pltpu
from jax.experimental.pallas import tpu_sc as plsc

_B = 4
_S = 4096
_D = 1024
_N = _B * _S
_NC = 2
_NS = 16
_NW = _NC * _NS
_PW = _S // _NW
_C = 16
_NCHUNK = _PW // _C
_L = 16


def _sc_kernel(tok_hbm, w_hbm, out_hbm, tok_v, idx_v, shared, wbuf0, wbuf1,
               rsem0, rsem1, gsem, osem0, osem1):
    wid = lax.axis_index("s") * _NC + lax.axis_index("c")
    sid = lax.axis_index("s")
    p0 = wid * _PW
    soff0 = sid * 2 * (_C + 1)
    soff1 = soff0 + (_C + 1)

    pltpu.sync_copy(w_hbm.at[pl.ds(0, 1)], shared.at[pl.ds(soff0, 1)])
    pltpu.sync_copy(w_hbm.at[pl.ds(0, 1)], shared.at[pl.ds(soff1, 1)])
    for b in range(_B):
        pltpu.sync_copy(tok_hbm.at[pl.ds(b * _S + p0, _PW)],
                        tok_v.at[pl.ds(b * _PW, _PW)])

    def build_idx(j, _):
        tok = tok_v[pl.ds(j * _L, _L)]
        local = jnp.where(tok != 0, lax.iota(jnp.int32, _L) + 1, 0)
        idx_v[pl.ds(j * _L, _L)] = local + soff0
        idx_v[pl.ds(_B * _PW + j * _L, _L)] = local + soff1
        return 0

    lax.fori_loop(0, (_B * _PW) // _L, build_idx, 0)

    soffs = (soff0, soff1)
    rsems = (rsem0, rsem1)
    wbufs = (wbuf0, wbuf1)
    osems = (osem0, osem1)

    def read(c, soff, rsem):
        return pltpu.async_copy(w_hbm.at[pl.ds(p0 + c * _C + 1, _C)],
                                shared.at[pl.ds(soff + 1, _C)], rsem)

    pending = [read(0, soff0, rsem0), read(1, soff1, rsem1)]
    out_pending = [None, None]
    for c in range(_NCHUNK):
        p = c % 2
        pending[p].wait()
        for b in range(_B):
            q = b % 2
            if out_pending[q] is not None:
                out_pending[q].wait()
            pltpu.async_copy(
                shared.at[idx_v.at[pl.ds(p * _B * _PW + b * _PW + c * _C,
                                         _C)]], wbufs[q], gsem).wait()
            out_pending[q] = pltpu.async_copy(
                wbufs[q], out_hbm.at[pl.ds(b * _S + p0 + c * _C, _C)],
                osems[q])
        if c + 2 < _NCHUNK:
            pending[p] = read(c + 2, soffs[p], rsems[p])
    out_pending[0].wait()
    out_pending[1].wait()


@jax.jit
def _run(tok_flat, weights):
    mesh = plsc.VectorSubcoreMesh(core_axis_name="c", subcore_axis_name="s")
    f = functools.partial(
        pl.kernel,
        mesh=mesh,
        out_type=jax.ShapeDtypeStruct((_N, 8, _D // 8), jnp.float32),
        scratch_types=[
            pltpu.VMEM((_B * _PW,), jnp.int32),
            pltpu.VMEM((2 * _B * _PW,), jnp.int32),
            pltpu.VMEM_SHARED((_NS * 2 * (_C + 1), 8, _D // 8), jnp.float32),
            pltpu.VMEM((_C, 8, _D // 8), jnp.float32),
            pltpu.VMEM((_C, 8, _D // 8), jnp.float32),
            pltpu.SemaphoreType.DMA,
            pltpu.SemaphoreType.DMA,
            pltpu.SemaphoreType.DMA,
            pltpu.SemaphoreType.DMA,
            pltpu.SemaphoreType.DMA,
        ],
    )(_sc_kernel)
    return f(tok_flat, weights)


def kernel(input, weights):
    tok_flat = input.reshape(-1)
    out = _run(tok_flat, weights.reshape(-1, 8, _D // 8))
    return out.reshape(_B, _S, _D)


# speedup vs baseline: 2.1308x; 2.1308x over previous
import functools

import jax
import jax.numpy as jnp
from jax import lax
from jax.experimental import pallas as pl
from jax.experimental.pallas import tpu as pltpu
from jax.experimental.pallas import tpu_sc as plsc

_B = 4
_S = 4096
_D = 1024
_N = _B * _S
_NC = 2
_NS = 16
_NW = _NC * _NS
_RW = _N // _NW
_C = 32
_NCHUNK = _RW // _C
_NB = 3
_L = 16


def _sc_kernel(tok_hbm, w_hbm, out_hbm, tok_v, idx_v, buf0, buf1, buf2,
               gsem0, gsem1, gsem2, osem0, osem1, osem2):
    wid = lax.axis_index("s") * _NC + lax.axis_index("c")
    base = wid * _RW
    pos0 = lax.rem(base, _S) + 1

    pltpu.sync_copy(tok_hbm.at[pl.ds(base, _RW)], tok_v)

    def build_idx(j, _):
        tok = tok_v[pl.ds(j * _L, _L)]
        ramp = lax.iota(jnp.int32, _L) + (pos0 + j * _L)
        idx_v[pl.ds(j * _L, _L)] = jnp.where(tok != 0, ramp, 0)
        return 0

    lax.fori_loop(0, _RW // _L, build_idx, 0)

    bufs = (buf0, buf1, buf2)
    gsems = (gsem0, gsem1, gsem2)
    osems = (osem0, osem1, osem2)

    def gather(c):
        p = c % _NB
        return pltpu.async_copy(w_hbm.at[idx_v.at[pl.ds(c * _C, _C)]],
                                bufs[p], gsems[p])

    pending = [gather(c) for c in range(_NB)]
    writes = [None] * _NCHUNK
    for c in range(_NCHUNK):
        p = c % _NB
        pending[p].wait()
        writes[c] = pltpu.async_copy(
            bufs[p], out_hbm.at[pl.ds(base + c * _C, _C)], osems[p])
        if c >= 1 and c + 2 < _NCHUNK:
            writes[c - 1].wait()
            writes[c - 1] = None
            pending[(c + 2) % _NB] = gather(c + 2)
    for wcp in writes:
        if wcp is not None:
            wcp.wait()


@jax.jit
def _run(tok_flat, weights):
    mesh = plsc.VectorSubcoreMesh(core_axis_name="c", subcore_axis_name="s")
    f = functools.partial(
        pl.kernel,
        mesh=mesh,
        out_type=jax.ShapeDtypeStruct((_N, _D), jnp.float32),
        scratch_types=[
            pltpu.VMEM((_RW,), jnp.int32),
            pltpu.VMEM((_RW,), jnp.int32),
            pltpu.VMEM((_C, _D), jnp.float32),
            pltpu.VMEM((_C, _D), jnp.float32),
            pltpu.VMEM((_C, _D), jnp.float32),
            pltpu.SemaphoreType.DMA,
            pltpu.SemaphoreType.DMA,
            pltpu.SemaphoreType.DMA,
            pltpu.SemaphoreType.DMA,
            pltpu.SemaphoreType.DMA,
            pltpu.SemaphoreType.DMA,
        ],
    )(_sc_kernel)
    return f(tok_flat, weights)


def kernel(input, weights):
    tok_flat = input.reshape(-1)
    out = _run(tok_flat, weights)
    return out.reshape(_B, _S, _D)
